# baseline (speedup 1.0000x reference)
import jax
import jax.numpy as jnp
from jax import lax
from jax.experimental import pallas as pl
from jax.experimental.pallas import tpu as pltpu

_TB = 256


def _acc_kernel(lens_ref, xl_ref, xh_ref, tgt_ref, out_ref, c_ref, m_ref):
    i = pl.program_id(0)
    nsteps = pl.num_programs(0)
    xl = xl_ref[...]
    xh = xh_ref[...]
    TB, B, H = xl.shape
    V = 2 * H
    rmax = jnp.maximum(
        jnp.max(xl, axis=-1, keepdims=True),
        jnp.max(xh, axis=-1, keepdims=True),
    )
    idx = lax.broadcasted_iota(jnp.int32, xl.shape, 2)
    pl_ = jnp.min(jnp.where(xl == rmax, idx, V), axis=-1)
    ph_ = jnp.min(jnp.where(xh == rmax, idx + H, V), axis=-1)
    pred = jnp.minimum(pl_, ph_)
    tgt = tgt_ref[0]
    t_idx = lax.broadcasted_iota(jnp.int32, (TB, B), 0) + i * _TB
    b_idx = lax.broadcasted_iota(jnp.int32, (TB, B), 1)
    lens_v = jnp.zeros((TB, B), jnp.int32)
    for b in range(B):
        lens_v = jnp.where(b_idx == b, lens_ref[b] + 1, lens_v)
    mask = t_idx < lens_v
    c_part = jnp.sum(jnp.where(mask & (pred == tgt), 1.0, 0.0))
    m_part = jnp.sum(jnp.where(mask, 1.0, 0.0))

    @pl.when(i == 0)
    def _init():
        c_ref[0] = 0.0
        m_ref[0] = 0.0

    c_ref[0] += c_part
    m_ref[0] += m_part

    @pl.when(i == nsteps - 1)
    def _fin():
        out_ref[0] = c_ref[0] / m_ref[0]


def kernel(outputs, tokens, tokens_lens):
    T, B, V = outputs.shape
    tgt = jnp.concatenate([tokens[1:], tokens[-1:]], axis=0)
    ntb = T // _TB
    tgt3 = tgt.reshape(ntb, _TB, B)
    grid_spec = pltpu.PrefetchScalarGridSpec(
        num_scalar_prefetch=1,
        grid=(ntb,),
        in_specs=[
            pl.BlockSpec((_TB, B, V // 2), lambda i, lens: (i, 0, 0)),
            pl.BlockSpec((_TB, B, V // 2), lambda i, lens: (i, 0, 1)),
            pl.BlockSpec((1, _TB, B), lambda i, lens: (i, 0, 0)),
        ],
        out_specs=pl.BlockSpec(memory_space=pltpu.SMEM),
        scratch_shapes=[
            pltpu.SMEM((1,), jnp.float32),
            pltpu.SMEM((1,), jnp.float32),
        ],
    )
    acc = pl.pallas_call(
        _acc_kernel,
        grid_spec=grid_spec,
        out_shape=jax.ShapeDtypeStruct((1,), jnp.float32),
        compiler_params=pltpu.CompilerParams(
            dimension_semantics=("arbitrary",),
        ),
    )(tokens_lens, outputs, outputs, tgt3)
    return acc[0]
